# Initial kernel scaffold; baseline (speedup 1.0000x reference)
import jax
import jax.numpy as jnp
from jax.experimental import pallas as pl


def kernel(x, edge_index, W_m, W_h, b_h, R_mean_w, R_mean_b, R_var_w, R_var_b):
    raise NotImplementedError("write your pallas kernel here")


# speedup vs baseline: 3.5401x; 3.5401x over previous
import functools

import jax
import jax.numpy as jnp
from jax import lax
from jax.experimental import pallas as pl
from jax.experimental.pallas import tpu as pltpu
from jax.experimental.pallas import tpu_sc as plsc

N, E, D, H, L = 10000, 160000, 256, 256, 64
DHALF = D // 2
NC, NS = 2, 16
CHUNK = 128
EPT = -(-E // (NS * CHUNK)) * CHUNK
EPAD = EPT * NS
NPAD = 10112
ZROWS = NPAD // NS

_mesh = plsc.VectorSubcoreMesh(core_axis_name="c", subcore_axis_name="s")


@functools.partial(
    pl.kernel,
    mesh=_mesh,
    out_type=[
        jax.ShapeDtypeStruct((NPAD, DHALF), jnp.float32),
        jax.ShapeDtypeStruct((NPAD, DHALF), jnp.float32),
    ],
    scratch_types=[
        pltpu.VMEM((CHUNK,), jnp.int32),
        pltpu.VMEM((CHUNK,), jnp.int32),
        pltpu.VMEM((CHUNK, DHALF), jnp.float32),
        pltpu.VMEM_SHARED((NPAD, DHALF), jnp.float32),
        pltpu.SemaphoreType.DMA,
    ],
)
def _sc_agg(xl_hbm, xr_hbm, src_hbm, dst_hbm, z0_hbm, outl_hbm, outr_hbm,
            src_v, dst_v, rows_v, acc_sh, sem):
    cid = lax.axis_index("c")
    sid = lax.axis_index("s")

    def run(xtab, out):
        pltpu.sync_copy(z0_hbm.at[pl.ds(sid * ZROWS, ZROWS)],
                        acc_sh.at[pl.ds(sid * ZROWS, ZROWS)])
        plsc.subcore_barrier()
        def body(i, _):
            e0 = sid * EPT + i * CHUNK
            pltpu.sync_copy(src_hbm.at[pl.ds(e0, CHUNK)], src_v)
            pltpu.sync_copy(dst_hbm.at[pl.ds(e0, CHUNK)], dst_v)
            pltpu.async_copy(xtab.at[src_v], rows_v, sem).wait()
            pltpu.sync_copy(rows_v, acc_sh.at[dst_v], add=True)
            return 0
        lax.fori_loop(0, EPT // CHUNK, body, 0)
        plsc.subcore_barrier()
        pltpu.sync_copy(acc_sh.at[pl.ds(sid * ZROWS, ZROWS)],
                        out.at[pl.ds(sid * ZROWS, ZROWS)])

    @pl.when(cid == 0)
    def _():
        run(xl_hbm, outl_hbm)

    @pl.when(cid == 1)
    def _():
        run(xr_hbm, outr_hbm)


def _dense_body(x_ref, agg_ref, wm_ref, whx_ref, wha_ref, bh_ref,
                rmw_ref, rmb_ref, rvw_ref, rvb_ref, eps_ref, z_ref):
    aggm = jnp.dot(agg_ref[...], wm_ref[...], preferred_element_type=jnp.float32)
    h = jnp.maximum(
        jnp.dot(x_ref[...], whx_ref[...], preferred_element_type=jnp.float32)
        + jnp.dot(aggm, wha_ref[...], preferred_element_type=jnp.float32)
        + bh_ref[...], 0.0)
    zm = jnp.dot(h, rmw_ref[...], preferred_element_type=jnp.float32) + rmb_ref[...]
    zlv = -jnp.abs(jnp.dot(h, rvw_ref[...], preferred_element_type=jnp.float32)
                   + rvb_ref[...])
    z_ref[...] = zm + jnp.exp(zlv * 0.5) * eps_ref[...]


_ROWS = 2000


def _dense(x, agg, W_m, whx, wha, b_h, R_mean_w, R_mean_b, R_var_w, R_var_b, eps):
    grid = (N // _ROWS,)
    full = lambda shape: pl.BlockSpec(shape, lambda i: (0, 0))
    return pl.pallas_call(
        _dense_body,
        grid=grid,
        in_specs=[
            pl.BlockSpec((_ROWS, D), lambda i: (i, 0)),
            pl.BlockSpec((_ROWS, D), lambda i: (i, 0)),
            full((D, H)),
            full((D, H)),
            full((H, H)),
            full((1, H)),
            full((H, L)),
            full((1, L)),
            full((H, L)),
            full((1, L)),
            pl.BlockSpec((_ROWS, L), lambda i: (i, 0)),
        ],
        out_specs=pl.BlockSpec((_ROWS, L), lambda i: (i, 0)),
        out_shape=jax.ShapeDtypeStruct((N, L), jnp.float32),
    )(x, agg, W_m, whx, wha, b_h.reshape(1, H),
      R_mean_w, R_mean_b.reshape(1, L), R_var_w, R_var_b.reshape(1, L), eps)


def kernel(x, edge_index, W_m, W_h, b_h, R_mean_w, R_mean_b, R_var_w, R_var_b):
    src = edge_index[0]
    dst = edge_index[1]
    pad = EPAD - E
    src_p = jnp.concatenate([src, jnp.zeros((pad,), jnp.int32)])
    dst_p = jnp.concatenate([dst, jnp.full((pad,), N, jnp.int32)])
    xl = x[:, :DHALF]
    xr = x[:, DHALF:]
    z0 = jnp.zeros((NPAD, DHALF), jnp.float32)
    aggl, aggr = _sc_agg(xl, xr, src_p, dst_p, z0)
    agg = jnp.concatenate([aggl[:N], aggr[:N]], axis=1)
    eps = jax.random.normal(jax.random.key(1), (N, L), dtype=jnp.float32)
    return _dense(x, agg, W_m, W_h[:D], W_h[D:], b_h,
                  R_mean_w, R_mean_b, R_var_w, R_var_b, eps)
